# Initial kernel scaffold; baseline (speedup 1.0000x reference)
import jax
import jax.numpy as jnp
from jax.experimental import pallas as pl


def kernel(features, gradients, memory_keys, memory_values, memory_ages, memory_surprise, W_key_enc, b_key_enc, W_val_enc, b_val_enc, W_proj, b_proj, W_q, b_q, W_k, b_k, W_v, b_v, W_ao, b_ao, W_op, b_op):
    raise NotImplementedError("write your pallas kernel here")


# speedup vs baseline: 1.4033x; 1.4033x over previous
import functools

import jax
import jax.numpy as jnp
import numpy as np
from jax.experimental import pallas as pl
from jax.experimental.pallas import tpu as pltpu

M = 65536
B = 128
D = 128
A = 136
H = 8
DH = A // H
THRESH = 0.1
DECAY = 0.95
TM = 512
HB = H * B


def _flash_kernel(f_ref, g_ref, mk_ref, mv_ref, mko_ref, mvo_ref,
                  wq2_ref, bq2_ref, wk2_ref, bk2_ref,
                  wke_ref, bke_ref, wve_ref, bve_ref,
                  wv2_ref, bv2_ref, wao2_ref, bao2_ref, hm_ref,
                  out_ref,
                  qm_s, qk_s, qbk_s, mrun_s, z_s, sraw_s, dmin_s):
    i = pl.program_id(0)
    nsteps = pl.num_programs(0)
    f = f_ref[...]

    @pl.when(i == 0)
    def _init():
        q = jnp.dot(f, wq2_ref[...], preferred_element_type=jnp.float32)
        q = (q + bq2_ref[...]) * np.float32(1.0 / np.sqrt(float(DH)))
        hm = hm_ref[...]
        qm = jnp.concatenate([q * hm[h:h + 1, :] for h in range(H)], axis=0)
        qm_s[...] = qm
        qk_s[...] = jax.lax.dot_general(
            qm, wk2_ref[...], (((1,), (1,)), ((), ())),
            preferred_element_type=jnp.float32)
        qbk_s[...] = jax.lax.dot_general(
            qm, bk2_ref[...], (((1,), (1,)), ((), ())),
            preferred_element_type=jnp.float32)
        mrun_s[...] = jnp.full((HB, 1), -1e30, jnp.float32)
        z_s[...] = jnp.zeros((HB, 1), jnp.float32)
        sraw_s[...] = jnp.zeros((HB, D), jnp.float32)
        dmin_s[...] = jnp.full((B, 1), 1e30, jnp.float32)

    mk = mk_ref[...]
    mv = mv_ref[...]

    s_bm = jax.lax.dot_general(f, mk, (((1,), (1,)), ((), ())),
                               preferred_element_type=jnp.float32)
    m2 = jax.lax.dot_general(jnp.ones((1, D), jnp.float32), mk * mk,
                             (((1,), (1,)), ((), ())),
                             preferred_element_type=jnp.float32)
    u = m2 - 2.0 * s_bm
    dmin_s[...] = jnp.minimum(dmin_s[...], jnp.min(u, axis=1, keepdims=True))

    s = jax.lax.dot_general(qk_s[...], mk, (((1,), (1,)), ((), ())),
                            preferred_element_type=jnp.float32) + qbk_s[...]
    tmax = jnp.max(s, axis=1, keepdims=True)
    mnew = jnp.maximum(mrun_s[...], tmax)
    alpha = jnp.exp(mrun_s[...] - mnew)
    p = jnp.exp(s - mnew)
    z_s[...] = z_s[...] * alpha + jnp.sum(p, axis=1, keepdims=True)
    sraw_s[...] = sraw_s[...] * alpha + jnp.dot(
        p, mv, preferred_element_type=jnp.float32)
    mrun_s[...] = mnew

    @pl.when(i == nsteps - 1)
    def _final():
        g = g_ref[...]
        g2 = jnp.sum(g * g, axis=1, keepdims=True)
        f2 = jnp.sum(f * f, axis=1, keepdims=True)
        d2min = jnp.maximum(f2 + dmin_s[...], 0.0)
        maskc = (g2 * d2min) > np.float32(THRESH * THRESH)

        enc_k = jnp.dot(f, wke_ref[...],
                        preferred_element_type=jnp.float32) + bke_ref[...]
        enc_v = jnp.dot(f, wve_ref[...],
                        preferred_element_type=jnp.float32) + bve_ref[...]
        mko = mko_ref[...]
        mvo = mvo_ref[...]
        upd_k = jnp.where(maskc, enc_k, mko)
        upd_v = jnp.where(maskc, enc_v, mvo)

        qk = qk_s[...]
        qbk = qbk_s[...]
        s_new = jax.lax.dot_general(qk, upd_k, (((1,), (1,)), ((), ())),
                                    preferred_element_type=jnp.float32) + qbk
        s_old = jax.lax.dot_general(qk, mko, (((1,), (1,)), ((), ())),
                                    preferred_element_type=jnp.float32) + qbk
        m1 = mrun_s[...]
        mg = jnp.maximum(m1, jnp.maximum(
            jnp.max(s_new, axis=1, keepdims=True),
            jnp.max(s_old, axis=1, keepdims=True)))
        a1 = jnp.exp(m1 - mg)
        p_new = jnp.exp(s_new - mg)
        p_old = jnp.exp(s_old - mg)
        z = (z_s[...] * a1
             + jnp.sum(p_new, axis=1, keepdims=True)
             - jnp.sum(p_old, axis=1, keepdims=True))
        sr = (sraw_s[...] * a1
              + jnp.dot(p_new, upd_v, preferred_element_type=jnp.float32)
              - jnp.dot(p_old, mvo, preferred_element_type=jnp.float32))
        ctx_stack = jnp.dot(sr / z, wv2_ref[...],
                            preferred_element_type=jnp.float32) + bv2_ref[...]
        hm = hm_ref[...]
        ctx = jnp.zeros((B, A), jnp.float32)
        for h in range(H):
            ctx = ctx + ctx_stack[h * B:(h + 1) * B, :] * hm[h:h + 1, :]
        out_ref[...] = jnp.dot(ctx, wao2_ref[...],
                               preferred_element_type=jnp.float32) + bao2_ref[...]


def kernel(features, gradients, memory_keys, memory_values, memory_ages,
           memory_surprise, W_key_enc, b_key_enc, W_val_enc, b_val_enc,
           W_proj, b_proj, W_q, b_q, W_k, b_k, W_v, b_v, W_ao, b_ao,
           W_op, b_op):
    evict_scores = memory_surprise * (DECAY ** memory_ages)
    _, slots = jax.lax.top_k(-evict_scores, B)
    mk_old = jnp.take(memory_keys, slots, axis=0)
    mv_old = jnp.take(memory_values, slots, axis=0)

    Wq2 = W_proj @ W_q
    bq2 = (b_proj @ W_q + b_q).reshape(1, A)
    Wk2 = W_proj @ W_k
    bk2 = (b_proj @ W_k + b_k).reshape(1, A)
    Wv2 = W_proj @ W_v
    bv2 = (b_proj @ W_v + b_v).reshape(1, A)
    Wao2 = W_ao @ W_op
    bao2 = (b_ao @ W_op + b_op).reshape(1, D)
    hm8 = jnp.repeat(jnp.eye(H, dtype=jnp.float32), DH, axis=1)

    nsteps = M // TM
    const = lambda i: (0, 0)
    in_specs = [
            pl.BlockSpec((B, D), const),
            pl.BlockSpec((B, D), const),
            pl.BlockSpec((TM, D), lambda i: (i, 0)),
            pl.BlockSpec((TM, D), lambda i: (i, 0)),
            pl.BlockSpec((B, D), const),
            pl.BlockSpec((B, D), const),
            pl.BlockSpec((D, A), const),
            pl.BlockSpec((1, A), const),
            pl.BlockSpec((D, A), const),
            pl.BlockSpec((1, A), const),
            pl.BlockSpec((D, D), const),
            pl.BlockSpec((1, D), const),
            pl.BlockSpec((D, D), const),
            pl.BlockSpec((1, D), const),
            pl.BlockSpec((D, A), const),
            pl.BlockSpec((1, A), const),
            pl.BlockSpec((A, D), const),
            pl.BlockSpec((1, D), const),
            pl.BlockSpec((H, A), const),
    ]
    retrieved = pl.pallas_call(
        _flash_kernel,
        grid=(nsteps,),
        in_specs=in_specs,
        out_specs=pl.BlockSpec((B, D), const),
        out_shape=jax.ShapeDtypeStruct((B, D), jnp.float32),
        scratch_shapes=[
            pltpu.VMEM((HB, A), jnp.float32),
            pltpu.VMEM((HB, D), jnp.float32),
            pltpu.VMEM((HB, 1), jnp.float32),
            pltpu.VMEM((HB, 1), jnp.float32),
            pltpu.VMEM((HB, 1), jnp.float32),
            pltpu.VMEM((HB, D), jnp.float32),
            pltpu.VMEM((B, 1), jnp.float32),
        ],
    )(features, gradients, memory_keys, memory_values, mk_old, mv_old,
      Wq2, bq2, Wk2, bk2, W_key_enc, b_key_enc.reshape(1, D),
      W_val_enc, b_val_enc.reshape(1, D), Wv2, bv2, Wao2, bao2, hm8)
    return retrieved
